# baseline (speedup 1.0000x reference)
import functools

import jax
import jax.numpy as jnp
from jax import lax
from jax.experimental import pallas as pl
from jax.experimental.pallas import tpu as pltpu
from jax.experimental.pallas import tpu_sc as plsc

_N = 10000
_E = 160000
_F = 128
_NC = 2
_NS = 16
_NW = _NC * _NS
_CH = 128
_NHALF = 2
_CPH = 20
_EPH = _CPH * _CH
_EPT = _NHALF * _EPH
_EP = _EPT * _NW
_NP = 10240
_RPT = _NP // _NS
_LANES = 16

_mesh = plsc.VectorSubcoreMesh(core_axis_name="c", subcore_axis_name="s")


@functools.partial(
    pl.kernel,
    out_type=jax.ShapeDtypeStruct((3, _NC, _NP, _F), jnp.float32),
    mesh=_mesh,
    scratch_types=[
        pltpu.VMEM((_EPH,), jnp.int32),
        pltpu.VMEM((_CPH, _CH), jnp.int32),
        pltpu.VMEM((_EPH,), jnp.float32),
        pltpu.VMEM((_CH, _F), jnp.float32),
        pltpu.VMEM((_CH, _F), jnp.float32),
        pltpu.VMEM_SHARED((_NP, _F), jnp.float32),
        pltpu.SemaphoreType.DMA,
        pltpu.SemaphoreType.DMA,
    ],
)
def _sc_spmm(x_hbm, src_hbm, dst_hbm, w_hbm, out_hbm,
             src_v, dst_v, w_v, rows0_v, rows1_v, acc_sh, sem_g0, sem_g1):
    cid = lax.axis_index("c")
    sid = lax.axis_index("s")
    wid = sid * _NC + cid
    row0 = sid * _RPT

    zv = jnp.zeros((_LANES,), jnp.float32)
    bufs = (rows0_v, rows1_v)
    sems = (sem_g0, sem_g1)

    for k in range(3):
        def _zfill(i, carry):
            for cb in range(_F // _LANES):
                rows0_v[i, pl.ds(cb * _LANES, _LANES)] = zv
            return carry

        lax.fori_loop(0, _CH, _zfill, 0)
        for r in range(_RPT // _CH):
            pltpu.sync_copy(rows0_v, acc_sh.at[pl.ds(row0 + r * _CH, _CH)])
        plsc.subcore_barrier()

        for hs in range(_NHALF):
            base = k * _EP + wid * _EPT + hs * _EPH
            pltpu.sync_copy(src_hbm.at[pl.ds(base, _EPH)], src_v)
            pltpu.sync_copy(w_hbm.at[pl.ds(base, _EPH)], w_v)
            pltpu.sync_copy(dst_hbm.at[k, wid, hs], dst_v)

            pltpu.async_copy(x_hbm.at[src_v.at[pl.ds(0, _CH)]], rows0_v, sem_g0)

            def pair_body(t, carry):
                for half in range(2):
                    j = 2 * t + half
                    rv, sg = bufs[half], sems[half]
                    ro, so = bufs[1 - half], sems[1 - half]
                    pltpu.make_async_copy(
                        x_hbm.at[src_v.at[pl.ds(j * _CH, _CH)]], rv, sg).wait()

                    @pl.when(j < _CPH - 1)
                    def _():
                        pltpu.async_copy(
                            x_hbm.at[src_v.at[pl.ds((j + 1) * _CH, _CH)]],
                            ro, so)

                    def g_body(g, c2):
                        w16 = w_v[pl.ds(j * _CH + g * _LANES, _LANES)]
                        for l in range(_LANES):
                            wsplat = w16.at[
                                jnp.full((_LANES,), l, jnp.int32)].get(
                                    mode="promise_in_bounds")
                            e = g * _LANES + l
                            for cb in range(_F // _LANES):
                                sl = pl.ds(cb * _LANES, _LANES)
                                rv[e, sl] = rv[e, sl] * wsplat
                        return c2

                    lax.fori_loop(0, _CH // _LANES, g_body, 0)

                    pltpu.sync_copy(rv, acc_sh.at[dst_v.at[j]], add=True)
                return carry

            lax.fori_loop(0, _CPH // 2, pair_body, 0)

        plsc.subcore_barrier()

        pltpu.sync_copy(acc_sh.at[pl.ds(row0, _RPT)],
                        out_hbm.at[k, cid, pl.ds(row0, _RPT)])


_BN = 1000


def _tc_finish_body(p_ref, wsub_ref, wcoef_ref, b_ref, out_ref):
    wsub = wsub_ref[...]
    hs = []
    for k in range(3):
        hk = p_ref[k, 0] + p_ref[k, 1]
        hs.append(lax.dot(hk, wsub, precision=lax.Precision.HIGHEST))
    c = hs[0] * hs[0] + hs[1] * hs[1] + hs[2] * hs[2]
    t = jnp.tanh(
        lax.dot(c, wcoef_ref[...], precision=lax.Precision.HIGHEST) + b_ref[...])
    out_ref[...] = jnp.stack([h * t for h in hs], axis=1)


def _tc_finish(partials, W_sub, W_coef, b_coef):
    return pl.pallas_call(
        _tc_finish_body,
        grid=(_N // _BN,),
        in_specs=[
            pl.BlockSpec((3, _NC, _BN, _F), lambda i: (0, 0, i, 0)),
            pl.BlockSpec((_F, _F), lambda i: (0, 0)),
            pl.BlockSpec((_F, _F), lambda i: (0, 0)),
            pl.BlockSpec((1, _F), lambda i: (0, 0)),
        ],
        out_specs=pl.BlockSpec((_BN, 3, _F), lambda i: (i, 0, 0)),
        out_shape=jax.ShapeDtypeStruct((_N, 3, _F), jnp.float32),
    )(partials, W_sub, W_coef, b_coef.reshape(1, _F))


def kernel(x, ei_x, ei_y, ei_z, w_x, w_y, w_z, W_sub, W_coef, b_coef):
    pad_i = jnp.zeros((_EP - _E,), jnp.int32)
    pad_f = jnp.zeros((_EP - _E,), jnp.float32)
    src = jnp.concatenate(
        [ei_x[1], pad_i, ei_y[1], pad_i, ei_z[1], pad_i])
    dst = jnp.concatenate(
        [ei_x[0], pad_i, ei_y[0], pad_i, ei_z[0], pad_i]).reshape(
            3, _NW, _NHALF, _CPH, _CH)
    w = jnp.concatenate([w_x, pad_f, w_y, pad_f, w_z, pad_f])
    partials = _sc_spmm(x, src, dst, w)
    return _tc_finish(partials, W_sub, W_coef, b_coef)


# speedup vs baseline: 2.4510x; 2.4510x over previous
import functools

import jax
import jax.numpy as jnp
from jax import lax
from jax.experimental import pallas as pl
from jax.experimental.pallas import tpu as pltpu
from jax.experimental.pallas import tpu_sc as plsc

_N = 10000
_E = 160000
_F = 128
_NC = 2
_NS = 16
_NW = _NC * _NS
_CH = 128
_NHALF = 2
_CPH = 20
_EPH = _CPH * _CH
_EPT = _NHALF * _EPH
_EP = _EPT * _NW
_NP = 10240
_RPT = _NP // _NS
_LANES = 16

_mesh = plsc.VectorSubcoreMesh(core_axis_name="c", subcore_axis_name="s")


@functools.partial(
    pl.kernel,
    out_type=jax.ShapeDtypeStruct((3, _NC, _NP, _F), jnp.float32),
    mesh=_mesh,
    scratch_types=[
        pltpu.VMEM((_EPH,), jnp.int32),
        pltpu.VMEM((_CPH, _CH), jnp.int32),
        pltpu.VMEM((_EPH,), jnp.float32),
        pltpu.VMEM((_CH, _F), jnp.float32),
        pltpu.VMEM((_CH, _F), jnp.float32),
        pltpu.VMEM_SHARED((_NP, _F), jnp.float32),
        pltpu.SemaphoreType.DMA,
        pltpu.SemaphoreType.DMA,
        pltpu.SemaphoreType.DMA,
        pltpu.SemaphoreType.DMA,
    ],
)
def _sc_spmm(x_hbm, src_hbm, dst_hbm, w_hbm, out_hbm,
             src_v, dst_v, w_v, rows0_v, rows1_v, acc_sh,
             sem_g0, sem_g1, sem_s0, sem_s1):
    cid = lax.axis_index("c")
    sid = lax.axis_index("s")
    wid = sid * _NC + cid
    row0 = sid * _RPT

    zv = jnp.zeros((_LANES,), jnp.float32)
    bufs = (rows0_v, rows1_v)
    gsems = (sem_g0, sem_g1)
    ssems = (sem_s0, sem_s1)

    for k in range(3):
        def _zfill(i, carry):
            for cb in range(_F // _LANES):
                rows0_v[i, pl.ds(cb * _LANES, _LANES)] = zv
            return carry

        lax.fori_loop(0, _CH, _zfill, 0)
        for r in range(_RPT // _CH):
            pltpu.sync_copy(rows0_v, acc_sh.at[pl.ds(row0 + r * _CH, _CH)])
        plsc.subcore_barrier()

        for hs in range(_NHALF):
            base = k * _EP + wid * _EPT + hs * _EPH
            pltpu.sync_copy(src_hbm.at[pl.ds(base, _EPH)], src_v)
            pltpu.sync_copy(w_hbm.at[pl.ds(base, _EPH)], w_v)
            pltpu.sync_copy(dst_hbm.at[k, wid, hs], dst_v)

            pltpu.async_copy(x_hbm.at[src_v.at[pl.ds(0, _CH)]], rows0_v, sem_g0)

            def pair_body(t, carry):
                for half in range(2):
                    j = 2 * t + half
                    rv, sg, ss = bufs[half], gsems[half], ssems[half]
                    ro, so, sso = bufs[1 - half], gsems[1 - half], ssems[1 - half]
                    pltpu.make_async_copy(
                        x_hbm.at[src_v.at[pl.ds(j * _CH, _CH)]], rv, sg).wait()

                    jm1 = jnp.maximum(j - 1, 0)

                    @pl.when(j > 0)
                    def _():
                        pltpu.make_async_copy(
                            ro, acc_sh.at[dst_v.at[jm1]], sso).wait()

                    @pl.when(j < _CPH - 1)
                    def _():
                        pltpu.async_copy(
                            x_hbm.at[src_v.at[pl.ds((j + 1) * _CH, _CH)]],
                            ro, so)

                    def g_body(g, c2):
                        w16 = w_v[pl.ds(j * _CH + g * _LANES, _LANES)]
                        for l in range(_LANES):
                            wsplat = w16.at[
                                jnp.full((_LANES,), l, jnp.int32)].get(
                                    mode="promise_in_bounds")
                            e = g * _LANES + l
                            for cb in range(_F // _LANES):
                                sl = pl.ds(cb * _LANES, _LANES)
                                rv[e, sl] = rv[e, sl] * wsplat
                        return c2

                    lax.fori_loop(0, _CH // _LANES, g_body, 0)

                    pltpu.async_copy(rv, acc_sh.at[dst_v.at[j]], ss, add=True)
                return carry

            lax.fori_loop(0, _CPH // 2, pair_body, 0)
            pltpu.make_async_copy(
                bufs[(_CPH - 1) % 2],
                acc_sh.at[dst_v.at[_CPH - 1]],
                ssems[(_CPH - 1) % 2]).wait()

        plsc.subcore_barrier()

        pltpu.sync_copy(acc_sh.at[pl.ds(row0, _RPT)],
                        out_hbm.at[k, cid, pl.ds(row0, _RPT)])


_BN = 1000


def _tc_finish_body(p_ref, wsub_ref, wcoef_ref, b_ref, out_ref):
    wsub = wsub_ref[...]
    hs = []
    for k in range(3):
        hk = p_ref[k, 0] + p_ref[k, 1]
        hs.append(lax.dot(hk, wsub, precision=lax.Precision.HIGHEST))
    c = hs[0] * hs[0] + hs[1] * hs[1] + hs[2] * hs[2]
    t = jnp.tanh(
        lax.dot(c, wcoef_ref[...], precision=lax.Precision.HIGHEST) + b_ref[...])
    out_ref[...] = jnp.stack([h * t for h in hs], axis=1)


def _tc_finish(partials, W_sub, W_coef, b_coef):
    return pl.pallas_call(
        _tc_finish_body,
        grid=(_N // _BN,),
        in_specs=[
            pl.BlockSpec((3, _NC, _BN, _F), lambda i: (0, 0, i, 0)),
            pl.BlockSpec((_F, _F), lambda i: (0, 0)),
            pl.BlockSpec((_F, _F), lambda i: (0, 0)),
            pl.BlockSpec((1, _F), lambda i: (0, 0)),
        ],
        out_specs=pl.BlockSpec((_BN, 3, _F), lambda i: (i, 0, 0)),
        out_shape=jax.ShapeDtypeStruct((_N, 3, _F), jnp.float32),
    )(partials, W_sub, W_coef, b_coef.reshape(1, _F))


def kernel(x, ei_x, ei_y, ei_z, w_x, w_y, w_z, W_sub, W_coef, b_coef):
    pad_i = jnp.arange(_EP - _E, dtype=jnp.int32) % _N
    pad_f = jnp.zeros((_EP - _E,), jnp.float32)
    src = jnp.concatenate(
        [ei_x[1], pad_i, ei_y[1], pad_i, ei_z[1], pad_i])
    dst = jnp.concatenate(
        [ei_x[0], pad_i, ei_y[0], pad_i, ei_z[0], pad_i]).reshape(
            3, _NW, _NHALF, _CPH, _CH)
    w = jnp.concatenate([w_x, pad_f, w_y, pad_f, w_z, pad_f])
    partials = _sc_spmm(x, src, dst, w)
    return _tc_finish(partials, W_sub, W_coef, b_coef)


# speedup vs baseline: 2.8049x; 1.1444x over previous
import functools

import jax
import jax.numpy as jnp
from jax import lax
from jax.experimental import pallas as pl
from jax.experimental.pallas import tpu as pltpu
from jax.experimental.pallas import tpu_sc as plsc

_N = 10000
_E = 160000
_F = 128
_NC = 2
_NS = 16
_NW = _NC * _NS
_CH = 128
_NCHUNKS = _E // _CH
_CPT = 40
_CPT_LAST = _NCHUNKS - (_NW - 1) * _CPT
_EPT = _CPT * _CH
_NP = 10240
_RPT = _NP // _NS
_LANES = 16

_mesh = plsc.VectorSubcoreMesh(core_axis_name="c", subcore_axis_name="s")


@functools.partial(
    pl.kernel,
    out_type=jax.ShapeDtypeStruct((3, _NC, _NP, _F), jnp.float32),
    mesh=_mesh,
    scratch_types=[
        pltpu.VMEM((_EPT,), jnp.int32),
        pltpu.VMEM((_CPT, _CH), jnp.int32),
        pltpu.VMEM((_EPT,), jnp.float32),
        pltpu.VMEM((_CH, _F), jnp.float32),
        pltpu.VMEM((_CH, _F), jnp.float32),
        pltpu.VMEM_SHARED((_NP, _F), jnp.float32),
        pltpu.SemaphoreType.DMA,
        pltpu.SemaphoreType.DMA,
        pltpu.SemaphoreType.DMA,
        pltpu.SemaphoreType.DMA,
    ],
)
def _sc_spmm(x_hbm, src_x_hbm, src_y_hbm, src_z_hbm,
             dst_x_hbm, dst_y_hbm, dst_z_hbm, w_x_hbm, w_y_hbm, w_z_hbm,
             out_hbm, src_v, dst_v, w_v, rows0_v, rows1_v, acc_sh,
             sem_g0, sem_g1, sem_s0, sem_s1):
    cid = lax.axis_index("c")
    sid = lax.axis_index("s")
    wid = sid * _NC + cid
    row0 = sid * _RPT
    last = wid == _NW - 1
    npair = jnp.where(last, _CPT_LAST // 2, _CPT // 2)

    zv = jnp.zeros((_LANES,), jnp.float32)
    bufs = (rows0_v, rows1_v)
    gsems = (sem_g0, sem_g1)
    ssems = (sem_s0, sem_s1)

    for k, (src_hbm, dst_hbm, wk_hbm) in enumerate((
            (src_x_hbm, dst_x_hbm, w_x_hbm),
            (src_y_hbm, dst_y_hbm, w_y_hbm),
            (src_z_hbm, dst_z_hbm, w_z_hbm))):
        def _zfill(i, carry):
            for cb in range(_F // _LANES):
                rows0_v[i, pl.ds(cb * _LANES, _LANES)] = zv
            return carry

        lax.fori_loop(0, _CH, _zfill, 0)
        for r in range(_RPT // _CH):
            pltpu.sync_copy(rows0_v, acc_sh.at[pl.ds(row0 + r * _CH, _CH)])

        pltpu.sync_copy(dst_hbm.at[wid], dst_v)

        @pl.when(jnp.logical_not(last))
        def _():
            pltpu.sync_copy(src_hbm.at[pl.ds(wid * _EPT, _EPT)], src_v)
            pltpu.sync_copy(wk_hbm.at[pl.ds(wid * _EPT, _EPT)], w_v)

        @pl.when(last)
        def _():
            nlast = _CPT_LAST * _CH
            sl = pl.ds(0, nlast)
            pltpu.sync_copy(
                src_hbm.at[pl.ds((_NW - 1) * _EPT, nlast)], src_v.at[sl])
            pltpu.sync_copy(
                wk_hbm.at[pl.ds((_NW - 1) * _EPT, nlast)], w_v.at[sl])

        plsc.subcore_barrier()

        pltpu.async_copy(x_hbm.at[src_v.at[pl.ds(0, _CH)]], rows0_v, sem_g0)

        def pair_body(t, carry):
            for half in range(2):
                j = 2 * t + half
                rv, sg, ss = bufs[half], gsems[half], ssems[half]
                ro, so, sso = bufs[1 - half], gsems[1 - half], ssems[1 - half]
                pltpu.make_async_copy(
                    x_hbm.at[src_v.at[pl.ds(j * _CH, _CH)]], rv, sg).wait()

                jm1 = jnp.maximum(j - 1, 0)

                @pl.when(j > 0)
                def _():
                    pltpu.make_async_copy(
                        ro, acc_sh.at[dst_v.at[jm1]], sso).wait()

                @pl.when(j + 1 < 2 * npair)
                def _():
                    pltpu.async_copy(
                        x_hbm.at[src_v.at[pl.ds((j + 1) * _CH, _CH)]], ro, so)

                def g_body(g, c2):
                    w16 = w_v[pl.ds(j * _CH + g * _LANES, _LANES)]
                    for l in range(_LANES):
                        wsplat = w16.at[
                            jnp.full((_LANES,), l, jnp.int32)].get(
                                mode="promise_in_bounds")
                        e = g * _LANES + l
                        for cb in range(_F // _LANES):
                            sl = pl.ds(cb * _LANES, _LANES)
                            rv[e, sl] = rv[e, sl] * wsplat
                    return c2

                lax.fori_loop(0, _CH // _LANES, g_body, 0)

                pltpu.async_copy(rv, acc_sh.at[dst_v.at[j]], ss, add=True)
            return carry

        lax.fori_loop(0, npair, pair_body, 0)
        jlast = 2 * npair - 1
        pltpu.make_async_copy(
            bufs[1], acc_sh.at[dst_v.at[jlast]], ssems[1]).wait()

        plsc.subcore_barrier()

        pltpu.sync_copy(acc_sh.at[pl.ds(row0, _RPT)],
                        out_hbm.at[k, cid, pl.ds(row0, _RPT)])


_BN = 2000


def _tc_finish_body(p_ref, wsub_ref, wcoef_ref, b_ref, out_ref):
    wsub = wsub_ref[...]
    hs = []
    for k in range(3):
        hk = p_ref[k, 0] + p_ref[k, 1]
        hs.append(lax.dot(hk, wsub, precision=lax.Precision.HIGHEST))
    c = hs[0] * hs[0] + hs[1] * hs[1] + hs[2] * hs[2]
    t = jnp.tanh(
        lax.dot(c, wcoef_ref[...], precision=lax.Precision.HIGHEST) + b_ref[...])
    out_ref[...] = jnp.stack([h * t for h in hs], axis=1)


def _tc_finish(partials, W_sub, W_coef, b_coef):
    return pl.pallas_call(
        _tc_finish_body,
        grid=(_N // _BN,),
        in_specs=[
            pl.BlockSpec((3, _NC, _BN, _F), lambda i: (0, 0, i, 0)),
            pl.BlockSpec((_F, _F), lambda i: (0, 0)),
            pl.BlockSpec((_F, _F), lambda i: (0, 0)),
            pl.BlockSpec((1, _F), lambda i: (0, 0)),
        ],
        out_specs=pl.BlockSpec((_BN, 3, _F), lambda i: (i, 0, 0)),
        out_shape=jax.ShapeDtypeStruct((_N, 3, _F), jnp.float32),
    )(partials, W_sub, W_coef, b_coef.reshape(1, _F))


def kernel(x, ei_x, ei_y, ei_z, w_x, w_y, w_z, W_sub, W_coef, b_coef):
    pad = jnp.zeros((_NW * _EPT - _E,), jnp.int32)

    def _dst2(ei):
        return jnp.concatenate([ei[0], pad]).reshape(_NW, _CPT, _CH)

    partials = _sc_spmm(
        x, ei_x[1], ei_y[1], ei_z[1],
        _dst2(ei_x), _dst2(ei_y), _dst2(ei_z), w_x, w_y, w_z)
    return _tc_finish(partials, W_sub, W_coef, b_coef)


# speedup vs baseline: 2.8680x; 1.0225x over previous
import functools

import jax
import jax.numpy as jnp
from jax import lax
from jax.experimental import pallas as pl
from jax.experimental.pallas import tpu as pltpu
from jax.experimental.pallas import tpu_sc as plsc

_N = 10000
_E = 160000
_F = 128
_NC = 2
_NS = 16
_NW = _NC * _NS
_CH = 128
_NCHUNKS = _E // _CH
_CPT = 40
_CPT_LAST = _NCHUNKS - (_NW - 1) * _CPT
_EPT = _CPT * _CH
_NP = 10240
_RPT = _NP // _NS
_LANES = 16

_mesh = plsc.VectorSubcoreMesh(core_axis_name="c", subcore_axis_name="s")


@functools.partial(
    pl.kernel,
    out_type=jax.ShapeDtypeStruct((3, _NC, _NP, _F), jnp.float32),
    mesh=_mesh,
    scratch_types=[
        pltpu.VMEM((_EPT,), jnp.int32),
        pltpu.VMEM((_CPT, _CH), jnp.int32),
        pltpu.VMEM((_EPT,), jnp.float32),
        pltpu.VMEM((_CH, _F), jnp.float32),
        pltpu.VMEM((_CH, _F), jnp.float32),
        pltpu.VMEM_SHARED((_NP, _F), jnp.float32),
        pltpu.SemaphoreType.DMA,
        pltpu.SemaphoreType.DMA,
        pltpu.SemaphoreType.DMA,
        pltpu.SemaphoreType.DMA,
        pltpu.SemaphoreType.DMA,
    ],
)
def _sc_spmm(x_hbm, src_x_hbm, src_y_hbm, src_z_hbm,
             dst_x_hbm, dst_y_hbm, dst_z_hbm, w_x_hbm, w_y_hbm, w_z_hbm,
             out_hbm, src_v, dst_v, w_v, rows0_v, rows1_v, acc_sh,
             sem_g0, sem_g1, sem_s0, sem_s1, sem_co):
    cid = lax.axis_index("c")
    sid = lax.axis_index("s")
    wid = sid * _NC + cid
    row0 = sid * _RPT
    last = wid == _NW - 1
    npair = jnp.where(last, _CPT_LAST // 2, _CPT // 2)

    zv = jnp.zeros((_LANES,), jnp.float32)
    bufs = (rows0_v, rows1_v)
    gsems = (sem_g0, sem_g1)
    ssems = (sem_s0, sem_s1)

    supports = ((src_x_hbm, dst_x_hbm, w_x_hbm),
                (src_y_hbm, dst_y_hbm, w_y_hbm),
                (src_z_hbm, dst_z_hbm, w_z_hbm))

    def _zfill_rows0():
        def _zf(i, carry):
            for cb in range(_F // _LANES):
                rows0_v[i, pl.ds(cb * _LANES, _LANES)] = zv
            return carry

        lax.fori_loop(0, _CH, _zf, 0)

    def _zero_acc():
        for r in range(_RPT // _CH):
            pltpu.sync_copy(rows0_v, acc_sh.at[pl.ds(row0 + r * _CH, _CH)])

    def _load_slabs(src_hbm, dst_hbm, wk_hbm):
        pltpu.sync_copy(dst_hbm.at[wid], dst_v)

        @pl.when(jnp.logical_not(last))
        def _():
            pltpu.sync_copy(src_hbm.at[pl.ds(wid * _EPT, _EPT)], src_v)
            pltpu.sync_copy(wk_hbm.at[pl.ds(wid * _EPT, _EPT)], w_v)

        @pl.when(last)
        def _():
            nlast = _CPT_LAST * _CH
            sl = pl.ds(0, nlast)
            pltpu.sync_copy(
                src_hbm.at[pl.ds((_NW - 1) * _EPT, nlast)], src_v.at[sl])
            pltpu.sync_copy(
                wk_hbm.at[pl.ds((_NW - 1) * _EPT, nlast)], w_v.at[sl])

    _zfill_rows0()
    _zero_acc()
    _load_slabs(*supports[0])
    plsc.subcore_barrier()

    for k in range(3):
        src_hbm, dst_hbm, wk_hbm = supports[k]
        pltpu.async_copy(x_hbm.at[src_v.at[pl.ds(0, _CH)]], rows0_v, sem_g0)

        def pair_body(t, carry):
            for half in range(2):
                j = 2 * t + half
                rv, sg, ss = bufs[half], gsems[half], ssems[half]
                ro, so, sso = bufs[1 - half], gsems[1 - half], ssems[1 - half]
                pltpu.make_async_copy(
                    x_hbm.at[src_v.at[pl.ds(j * _CH, _CH)]], rv, sg).wait()

                jm1 = jnp.maximum(j - 1, 0)

                @pl.when(j > 0)
                def _():
                    pltpu.make_async_copy(
                        ro, acc_sh.at[dst_v.at[jm1]], sso).wait()

                @pl.when(j + 1 < 2 * npair)
                def _():
                    pltpu.async_copy(
                        x_hbm.at[src_v.at[pl.ds((j + 1) * _CH, _CH)]], ro, so)

                def g_body(g, c2):
                    w16 = w_v[pl.ds(j * _CH + g * _LANES, _LANES)]
                    for l in range(_LANES):
                        wsplat = w16.at[
                            jnp.full((_LANES,), l, jnp.int32)].get(
                                mode="promise_in_bounds")
                        e = g * _LANES + l
                        for cb in range(_F // _LANES):
                            sl = pl.ds(cb * _LANES, _LANES)
                            rv[e, sl] = rv[e, sl] * wsplat
                    return c2

                lax.fori_loop(0, _CH // _LANES, g_body, 0)

                pltpu.async_copy(rv, acc_sh.at[dst_v.at[j]], ss, add=True)
            return carry

        lax.fori_loop(0, npair, pair_body, 0)
        jlast = 2 * npair - 1
        pltpu.make_async_copy(
            bufs[1], acc_sh.at[dst_v.at[jlast]], ssems[1]).wait()

        plsc.subcore_barrier()

        out_slice = out_hbm.at[k, cid, pl.ds(row0, _RPT)]
        pltpu.async_copy(acc_sh.at[pl.ds(row0, _RPT)], out_slice, sem_co)
        if k < 2:
            _zfill_rows0()
            _load_slabs(*supports[k + 1])
            pltpu.make_async_copy(
                acc_sh.at[pl.ds(row0, _RPT)], out_slice, sem_co).wait()
            _zero_acc()
            plsc.subcore_barrier()
        else:
            pltpu.make_async_copy(
                acc_sh.at[pl.ds(row0, _RPT)], out_slice, sem_co).wait()


_BN = 2000


def _tc_finish_body(p_ref, wsub_ref, wcoef_ref, b_ref, out_ref):
    wsub = wsub_ref[...]
    hs = []
    for k in range(3):
        hk = p_ref[k, 0] + p_ref[k, 1]
        hs.append(lax.dot(hk, wsub, precision=lax.Precision.HIGHEST))
    c = hs[0] * hs[0] + hs[1] * hs[1] + hs[2] * hs[2]
    t = jnp.tanh(
        lax.dot(c, wcoef_ref[...], precision=lax.Precision.HIGHEST) + b_ref[...])
    out_ref[...] = jnp.stack([h * t for h in hs], axis=1)


def _tc_finish(partials, W_sub, W_coef, b_coef):
    return pl.pallas_call(
        _tc_finish_body,
        grid=(_N // _BN,),
        in_specs=[
            pl.BlockSpec((3, _NC, _BN, _F), lambda i: (0, 0, i, 0)),
            pl.BlockSpec((_F, _F), lambda i: (0, 0)),
            pl.BlockSpec((_F, _F), lambda i: (0, 0)),
            pl.BlockSpec((1, _F), lambda i: (0, 0)),
        ],
        out_specs=pl.BlockSpec((_BN, 3, _F), lambda i: (i, 0, 0)),
        out_shape=jax.ShapeDtypeStruct((_N, 3, _F), jnp.float32),
    )(partials, W_sub, W_coef, b_coef.reshape(1, _F))


def kernel(x, ei_x, ei_y, ei_z, w_x, w_y, w_z, W_sub, W_coef, b_coef):
    pad = jnp.zeros((_NW * _EPT - _E,), jnp.int32)

    def _dst2(ei):
        return jnp.concatenate([ei[0], pad]).reshape(_NW, _CPT, _CH)

    partials = _sc_spmm(
        x, ei_x[1], ei_y[1], ei_z[1],
        _dst2(ei_x), _dst2(ei_y), _dst2(ei_z), w_x, w_y, w_z)
    return _tc_finish(partials, W_sub, W_coef, b_coef)


# speedup vs baseline: 2.9037x; 1.0124x over previous
import functools

import jax
import jax.numpy as jnp
from jax import lax
from jax.experimental import pallas as pl
from jax.experimental.pallas import tpu as pltpu
from jax.experimental.pallas import tpu_sc as plsc

_N = 10000
_E = 160000
_F = 128
_NC = 2
_NS = 16
_NW = _NC * _NS
_CH = 128
_NCHUNKS = _E // _CH
_CPT = 40
_CPT_LAST = _NCHUNKS - (_NW - 1) * _CPT
_EPT = _CPT * _CH
_NP = 10240
_RPT = _NP // _NS
_LANES = 16

_mesh = plsc.VectorSubcoreMesh(core_axis_name="c", subcore_axis_name="s")


@functools.partial(
    pl.kernel,
    out_type=jax.ShapeDtypeStruct((3, _NC, _NP, _F), jnp.float32),
    mesh=_mesh,
    scratch_types=[
        pltpu.VMEM((_EPT,), jnp.int32),
        pltpu.VMEM((_CPT, _CH), jnp.int32),
        pltpu.VMEM((_EPT,), jnp.float32),
        pltpu.VMEM((_CH, _F), jnp.float32),
        pltpu.VMEM((_CH, _F), jnp.float32),
        pltpu.VMEM_SHARED((_NP, _F), jnp.float32),
        pltpu.SemaphoreType.DMA,
        pltpu.SemaphoreType.DMA,
        pltpu.SemaphoreType.DMA,
        pltpu.SemaphoreType.DMA,
        pltpu.SemaphoreType.DMA,
    ],
)
def _sc_spmm(x_hbm, src_x_hbm, src_y_hbm, src_z_hbm,
             dst_x_hbm, dst_y_hbm, dst_z_hbm, w_x_hbm, w_y_hbm, w_z_hbm,
             out_hbm, src_v, dst_v, w_v, rows0_v, rows1_v, acc_sh,
             sem_g0, sem_g1, sem_s0, sem_s1, sem_co):
    cid = lax.axis_index("c")
    sid = lax.axis_index("s")
    wid = sid * _NC + cid
    row0 = sid * _RPT
    last = wid == _NW - 1
    npair = jnp.where(last, _CPT_LAST // 2, _CPT // 2)

    zv = jnp.zeros((_LANES,), jnp.float32)
    bufs = (rows0_v, rows1_v)
    gsems = (sem_g0, sem_g1)
    ssems = (sem_s0, sem_s1)

    supports = ((src_x_hbm, dst_x_hbm, w_x_hbm),
                (src_y_hbm, dst_y_hbm, w_y_hbm),
                (src_z_hbm, dst_z_hbm, w_z_hbm))

    def _zfill_rows0():
        def _zf(i, carry):
            for cb in range(_F // _LANES):
                rows0_v[i, pl.ds(cb * _LANES, _LANES)] = zv
            return carry

        lax.fori_loop(0, _CH, _zf, 0)

    def _zero_acc():
        for r in range(_RPT // _CH):
            pltpu.sync_copy(rows0_v, acc_sh.at[pl.ds(row0 + r * _CH, _CH)])

    def _load_slabs(src_hbm, dst_hbm, wk_hbm):
        pltpu.sync_copy(dst_hbm.at[wid], dst_v)

        @pl.when(jnp.logical_not(last))
        def _():
            pltpu.sync_copy(src_hbm.at[pl.ds(wid * _EPT, _EPT)], src_v)
            pltpu.sync_copy(wk_hbm.at[pl.ds(wid * _EPT, _EPT)], w_v)

        @pl.when(last)
        def _():
            nlast = _CPT_LAST * _CH
            sl = pl.ds(0, nlast)
            pltpu.sync_copy(
                src_hbm.at[pl.ds((_NW - 1) * _EPT, nlast)], src_v.at[sl])
            pltpu.sync_copy(
                wk_hbm.at[pl.ds((_NW - 1) * _EPT, nlast)], w_v.at[sl])

    _zfill_rows0()
    _zero_acc()
    _load_slabs(*supports[0])
    plsc.subcore_barrier()

    for k in range(3):
        src_hbm, dst_hbm, wk_hbm = supports[k]
        pltpu.async_copy(x_hbm.at[src_v.at[pl.ds(0, _CH)]], rows0_v, sem_g0)

        def pair_body(t, carry):
            for half in range(2):
                j = 2 * t + half
                rv, sg, ss = bufs[half], gsems[half], ssems[half]
                ro, so, sso = bufs[1 - half], gsems[1 - half], ssems[1 - half]
                jm1 = jnp.maximum(j - 1, 0)

                @pl.when(j > 0)
                def _():
                    pltpu.make_async_copy(
                        ro, acc_sh.at[dst_v.at[jm1]], sso).wait()

                @pl.when(j + 1 < 2 * npair)
                def _():
                    pltpu.async_copy(
                        x_hbm.at[src_v.at[pl.ds((j + 1) * _CH, _CH)]], ro, so)

                pltpu.make_async_copy(
                    x_hbm.at[src_v.at[pl.ds(j * _CH, _CH)]], rv, sg).wait()

                def g_body(g, c2):
                    w16 = w_v[pl.ds(j * _CH + g * _LANES, _LANES)]
                    for l in range(_LANES):
                        wsplat = w16.at[
                            jnp.full((_LANES,), l, jnp.int32)].get(
                                mode="promise_in_bounds")
                        e = g * _LANES + l
                        for cb in range(_F // _LANES):
                            sl = pl.ds(cb * _LANES, _LANES)
                            rv[e, sl] = rv[e, sl] * wsplat
                    return c2

                lax.fori_loop(0, _CH // _LANES, g_body, 0)

                pltpu.async_copy(rv, acc_sh.at[dst_v.at[j]], ss, add=True)
            return carry

        lax.fori_loop(0, npair, pair_body, 0)
        jlast = 2 * npair - 1
        pltpu.make_async_copy(
            bufs[1], acc_sh.at[dst_v.at[jlast]], ssems[1]).wait()

        plsc.subcore_barrier()

        out_slice = out_hbm.at[k, cid, pl.ds(row0, _RPT)]
        pltpu.async_copy(acc_sh.at[pl.ds(row0, _RPT)], out_slice, sem_co)
        if k < 2:
            _zfill_rows0()
            _load_slabs(*supports[k + 1])
            pltpu.make_async_copy(
                acc_sh.at[pl.ds(row0, _RPT)], out_slice, sem_co).wait()
            _zero_acc()
            plsc.subcore_barrier()
        else:
            pltpu.make_async_copy(
                acc_sh.at[pl.ds(row0, _RPT)], out_slice, sem_co).wait()


_BN = 2000


def _tc_finish_body(p_ref, wsub_ref, wcoef_ref, b_ref, out_ref):
    wsub = wsub_ref[...]
    hs = []
    for k in range(3):
        hk = p_ref[k, 0] + p_ref[k, 1]
        hs.append(lax.dot(hk, wsub, precision=lax.Precision.HIGHEST))
    c = hs[0] * hs[0] + hs[1] * hs[1] + hs[2] * hs[2]
    t = jnp.tanh(
        lax.dot(c, wcoef_ref[...], precision=lax.Precision.HIGHEST) + b_ref[...])
    out_ref[...] = jnp.stack([h * t for h in hs], axis=1)


def _tc_finish(partials, W_sub, W_coef, b_coef):
    return pl.pallas_call(
        _tc_finish_body,
        grid=(_N // _BN,),
        in_specs=[
            pl.BlockSpec((3, _NC, _BN, _F), lambda i: (0, 0, i, 0)),
            pl.BlockSpec((_F, _F), lambda i: (0, 0)),
            pl.BlockSpec((_F, _F), lambda i: (0, 0)),
            pl.BlockSpec((1, _F), lambda i: (0, 0)),
        ],
        out_specs=pl.BlockSpec((_BN, 3, _F), lambda i: (i, 0, 0)),
        out_shape=jax.ShapeDtypeStruct((_N, 3, _F), jnp.float32),
    )(partials, W_sub, W_coef, b_coef.reshape(1, _F))


def kernel(x, ei_x, ei_y, ei_z, w_x, w_y, w_z, W_sub, W_coef, b_coef):
    pad = jnp.zeros((_NW * _EPT - _E,), jnp.int32)

    def _dst2(ei):
        return jnp.concatenate([ei[0], pad]).reshape(_NW, _CPT, _CH)

    partials = _sc_spmm(
        x, ei_x[1], ei_y[1], ei_z[1],
        _dst2(ei_x), _dst2(ei_y), _dst2(ei_z), w_x, w_y, w_z)
    return _tc_finish(partials, W_sub, W_coef, b_coef)


# speedup vs baseline: 2.9290x; 1.0087x over previous
import functools

import jax
import jax.numpy as jnp
from jax import lax
from jax.experimental import pallas as pl
from jax.experimental.pallas import tpu as pltpu
from jax.experimental.pallas import tpu_sc as plsc

_N = 10000
_E = 160000
_F = 128
_NC = 2
_NS = 16
_NW = _NC * _NS
_CH = 128
_NCHUNKS = _E // _CH
_CPT = 40
_CPT_LAST = _NCHUNKS - (_NW - 1) * _CPT
_EPT = _CPT * _CH
_NP = 10240
_RPT = _NP // _NS
_LANES = 16

_mesh = plsc.VectorSubcoreMesh(core_axis_name="c", subcore_axis_name="s")


@functools.partial(
    pl.kernel,
    out_type=jax.ShapeDtypeStruct((3, _NC, _NP, _F), jnp.float32),
    mesh=_mesh,
    scratch_types=[
        pltpu.VMEM((_EPT,), jnp.int32),
        pltpu.VMEM((_CPT, _CH), jnp.int32),
        pltpu.VMEM((_EPT,), jnp.float32),
        pltpu.VMEM((_CH, _F), jnp.float32),
        pltpu.VMEM((_CH, _F), jnp.float32),
        pltpu.VMEM_SHARED((_NP, _F), jnp.float32),
        pltpu.SemaphoreType.DMA,
        pltpu.SemaphoreType.DMA,
        pltpu.SemaphoreType.DMA,
        pltpu.SemaphoreType.DMA,
        pltpu.SemaphoreType.DMA,
    ],
)
def _sc_spmm(x_hbm, src_x_hbm, src_y_hbm, src_z_hbm,
             dst_x_hbm, dst_y_hbm, dst_z_hbm, w_x_hbm, w_y_hbm, w_z_hbm,
             out_hbm, src_v, dst_v, w_v, rows0_v, rows1_v, acc_sh,
             sem_g0, sem_g1, sem_s0, sem_s1, sem_co):
    cid = lax.axis_index("c")
    sid = lax.axis_index("s")
    wid = sid * _NC + cid
    row0 = sid * _RPT
    last = wid == _NW - 1
    npair = jnp.where(last, _CPT_LAST // 2, _CPT // 2)

    zv = jnp.zeros((_LANES,), jnp.float32)
    bufs = (rows0_v, rows1_v)
    gsems = (sem_g0, sem_g1)
    ssems = (sem_s0, sem_s1)

    supports = ((src_x_hbm, dst_x_hbm, w_x_hbm),
                (src_y_hbm, dst_y_hbm, w_y_hbm),
                (src_z_hbm, dst_z_hbm, w_z_hbm))

    def _zfill_rows0():
        def _zf(i, carry):
            for cb in range(_F // _LANES):
                rows0_v[i, pl.ds(cb * _LANES, _LANES)] = zv
            return carry

        lax.fori_loop(0, _CH, _zf, 0)

    def _zero_acc():
        for r in range(_RPT // _CH):
            pltpu.sync_copy(rows0_v, acc_sh.at[pl.ds(row0 + r * _CH, _CH)])

    def _load_slabs(src_hbm, dst_hbm, wk_hbm):
        pltpu.sync_copy(dst_hbm.at[wid], dst_v)

        @pl.when(jnp.logical_not(last))
        def _():
            pltpu.sync_copy(src_hbm.at[pl.ds(wid * _EPT, _EPT)], src_v)
            pltpu.sync_copy(wk_hbm.at[pl.ds(wid * _EPT, _EPT)], w_v)

        @pl.when(last)
        def _():
            nlast = _CPT_LAST * _CH
            sl = pl.ds(0, nlast)
            pltpu.sync_copy(
                src_hbm.at[pl.ds((_NW - 1) * _EPT, nlast)], src_v.at[sl])
            pltpu.sync_copy(
                wk_hbm.at[pl.ds((_NW - 1) * _EPT, nlast)], w_v.at[sl])

    def _prime():
        pltpu.async_copy(x_hbm.at[src_v.at[pl.ds(0, _CH)]], rows0_v, sem_g0)

    _zfill_rows0()
    _zero_acc()
    _load_slabs(*supports[0])
    _prime()
    plsc.subcore_barrier()

    for k in range(3):
        src_hbm, dst_hbm, wk_hbm = supports[k]

        def pair_body(t, carry):
            for half in range(2):
                j = 2 * t + half
                rv, sg, ss = bufs[half], gsems[half], ssems[half]
                ro, so, sso = bufs[1 - half], gsems[1 - half], ssems[1 - half]
                jm1 = jnp.maximum(j - 1, 0)

                @pl.when(j > 0)
                def _():
                    pltpu.make_async_copy(
                        ro, acc_sh.at[dst_v.at[jm1]], sso).wait()

                @pl.when(j + 1 < 2 * npair)
                def _():
                    pltpu.async_copy(
                        x_hbm.at[src_v.at[pl.ds((j + 1) * _CH, _CH)]], ro, so)

                pltpu.make_async_copy(
                    x_hbm.at[src_v.at[pl.ds(j * _CH, _CH)]], rv, sg).wait()

                def g_body(g, c2):
                    w16 = w_v[pl.ds(j * _CH + g * _LANES, _LANES)]
                    for l in range(_LANES):
                        wsplat = w16.at[
                            jnp.full((_LANES,), l, jnp.int32)].get(
                                mode="promise_in_bounds")
                        e = g * _LANES + l
                        for cb in range(_F // _LANES):
                            sl = pl.ds(cb * _LANES, _LANES)
                            rv[e, sl] = rv[e, sl] * wsplat
                    return c2

                lax.fori_loop(0, _CH // _LANES, g_body, 0)

                pltpu.async_copy(rv, acc_sh.at[dst_v.at[j]], ss, add=True)
            return carry

        lax.fori_loop(0, npair, pair_body, 0)
        jlast = 2 * npair - 1
        pltpu.make_async_copy(
            bufs[1], acc_sh.at[dst_v.at[jlast]], ssems[1]).wait()

        plsc.subcore_barrier()

        out_slice = out_hbm.at[k, cid, pl.ds(row0, _RPT)]
        pltpu.async_copy(acc_sh.at[pl.ds(row0, _RPT)], out_slice, sem_co)
        if k < 2:
            _zfill_rows0()
            _load_slabs(*supports[k + 1])
            pltpu.make_async_copy(
                acc_sh.at[pl.ds(row0, _RPT)], out_slice, sem_co).wait()
            _zero_acc()
            _prime()
            plsc.subcore_barrier()
        else:
            pltpu.make_async_copy(
                acc_sh.at[pl.ds(row0, _RPT)], out_slice, sem_co).wait()


_BN = 2000


def _tc_finish_body(p_ref, wsub_ref, wcoef_ref, b_ref, out_ref):
    wsub = wsub_ref[...]
    hs = []
    for k in range(3):
        hk = p_ref[k, 0] + p_ref[k, 1]
        hs.append(lax.dot(hk, wsub, precision=lax.Precision.HIGHEST))
    c = hs[0] * hs[0] + hs[1] * hs[1] + hs[2] * hs[2]
    t = jnp.tanh(
        lax.dot(c, wcoef_ref[...], precision=lax.Precision.HIGHEST) + b_ref[...])
    out_ref[...] = jnp.stack([h * t for h in hs], axis=1)


def _tc_finish(partials, W_sub, W_coef, b_coef):
    return pl.pallas_call(
        _tc_finish_body,
        grid=(_N // _BN,),
        in_specs=[
            pl.BlockSpec((3, _NC, _BN, _F), lambda i: (0, 0, i, 0)),
            pl.BlockSpec((_F, _F), lambda i: (0, 0)),
            pl.BlockSpec((_F, _F), lambda i: (0, 0)),
            pl.BlockSpec((1, _F), lambda i: (0, 0)),
        ],
        out_specs=pl.BlockSpec((_BN, 3, _F), lambda i: (i, 0, 0)),
        out_shape=jax.ShapeDtypeStruct((_N, 3, _F), jnp.float32),
    )(partials, W_sub, W_coef, b_coef.reshape(1, _F))


def kernel(x, ei_x, ei_y, ei_z, w_x, w_y, w_z, W_sub, W_coef, b_coef):
    pad = jnp.zeros((_NW * _EPT - _E,), jnp.int32)

    def _dst2(ei):
        return jnp.concatenate([ei[0], pad]).reshape(_NW, _CPT, _CH)

    partials = _sc_spmm(
        x, ei_x[1], ei_y[1], ei_z[1],
        _dst2(ei_x), _dst2(ei_y), _dst2(ei_z), w_x, w_y, w_z)
    return _tc_finish(partials, W_sub, W_coef, b_coef)
